# baseline (speedup 1.0000x reference)
import functools

import jax
import jax.numpy as jnp
from jax import lax
from jax.experimental import pallas as pl
from jax.experimental.pallas import tpu as pltpu
from jax.experimental.pallas import tpu_sc as plsc

N = 10000
E = 320000
D = 128
NPAD = 10240
NW = 32
EW = E // NW
CH = 128
NCH = 80
EPAD = NW * NCH * CH
RPW = NPAD // 16

_mesh = plsc.VectorSubcoreMesh(core_axis_name="c", subcore_axis_name="s")
_sc_params = pltpu.CompilerParams(needs_layout_passes=False)


@functools.partial(
    pl.kernel,
    out_type=jax.ShapeDtypeStruct((NW, NPAD), jnp.int32),
    mesh=_mesh,
    scratch_types=[
        pltpu.VMEM((EW,), jnp.int32),
        pltpu.VMEM((NPAD,), jnp.int32),
    ],
    compiler_params=_sc_params,
)
def _sc_hist(row_hbm, out_hbm, idx_v, hist_v):
    cid = lax.axis_index("c")
    sid = lax.axis_index("s")
    w = cid * 16 + sid
    pltpu.sync_copy(row_hbm.at[pl.ds(w * EW, EW)], idx_v)

    def zero_body(i, c):
        hist_v[pl.ds(i * 16, 16)] = jnp.zeros((16,), jnp.int32)
        return c

    lax.fori_loop(0, NPAD // 16, zero_body, 0)
    ones = jnp.ones((16,), jnp.int32)

    def body(i, c):
        idx = idx_v[pl.ds(i * 16, 16)]
        plsc.addupdate_scatter(hist_v, [idx], ones)
        return c

    lax.fori_loop(0, EW // 16, body, 0)
    pltpu.sync_copy(hist_v, out_hbm.at[w])


@functools.partial(
    pl.kernel,
    out_type=jax.ShapeDtypeStruct((2, NPAD, D), jnp.float32),
    mesh=_mesh,
    scratch_types=[
        pltpu.VMEM((NCH // 2, CH), jnp.int32),
        pltpu.VMEM((NCH // 2, CH), jnp.int32),
        pltpu.VMEM((CH, D), jnp.float32),
        pltpu.VMEM((CH, D), jnp.float32),
        pltpu.VMEM_SHARED((NPAD, D), jnp.float32),
        pltpu.SemaphoreType.DMA,
        pltpu.SemaphoreType.DMA,
        pltpu.SemaphoreType.DMA,
        pltpu.SemaphoreType.DMA,
    ],
    compiler_params=_sc_params,
)
def _sc_spmm(y_hbm, col_hbm, row_hbm, out_hbm, col_v, row_v, rows0, rows1,
             agg_sh, sg0, sg1, ss0, ss1):
    cid = lax.axis_index("c")
    sid = lax.axis_index("s")
    w = cid * 16 + sid
    bufs = (rows0, rows1)
    sgs = (sg0, sg1)
    sss = (ss0, ss1)
    HC = NCH // 2

    def zb(i, c):
        rows0[i // 8, pl.ds((i % 8) * 16, 16)] = jnp.zeros((16,), jnp.float32)
        return c

    lax.fori_loop(0, CH * (D // 16), zb, 0)
    zcps = [
        pltpu.async_copy(rows0, agg_sh.at[pl.ds(sid * RPW + i * CH, CH)], sg0)
        for i in range(RPW // CH)
    ]
    for cp in zcps:
        cp.wait()
    plsc.subcore_barrier()

    for h in range(2):
        pltpu.sync_copy(col_hbm.at[w, pl.ds(h * HC, HC)], col_v)
        pltpu.sync_copy(row_hbm.at[w, pl.ds(h * HC, HC)], row_v)
        g0 = pltpu.async_copy(y_hbm.at[col_v.at[0]], rows0, sg0)
        g1 = pltpu.async_copy(y_hbm.at[col_v.at[1]], rows1, sg1)
        gathers = [g0, g1]

        def pair(g, c):
            j = 2 * g
            gathers[0].wait()
            gathers[1].wait()
            s0 = pltpu.async_copy(
                bufs[0], agg_sh.at[row_v.at[j]], sss[0], add=True)
            s1 = pltpu.async_copy(
                bufs[1], agg_sh.at[row_v.at[j + 1]], sss[1], add=True)
            s0.wait()

            @pl.when(j + 2 < HC)
            def _():
                pltpu.async_copy(y_hbm.at[col_v.at[j + 2]], bufs[0], sgs[0])

            s1.wait()

            @pl.when(j + 3 < HC)
            def _():
                pltpu.async_copy(y_hbm.at[col_v.at[j + 3]], bufs[1], sgs[1])

            return c

        lax.fori_loop(0, HC // 2, pair, 0)

    plsc.subcore_barrier()
    wcps = [
        pltpu.async_copy(agg_sh.at[pl.ds(sid * RPW + i * CH, CH)],
                         out_hbm.at[cid, pl.ds(sid * RPW + i * CH, CH)], sg0)
        for i in range(RPW // CH)
    ]
    for cp in wcps:
        cp.wait()


def _tc_scale_body(hist_ref, x_ref, y_ref):
    deg = jnp.sum(hist_ref[...], axis=0).astype(jnp.float32) + 1.0
    dinv = lax.rsqrt(deg)
    y_ref[pl.ds(0, N)] = x_ref[...] * dinv[:N, None]
    y_ref[pl.ds(N, NPAD - N)] = jnp.zeros((NPAD - N, D), jnp.float32)


def _tc_final_body(p_ref, x_ref, hist_ref, w_ref, g_ref, b_ref, o_ref):
    deg = (jnp.sum(hist_ref[...], axis=0).astype(jnp.float32) + 1.0)[:N]
    dinv = lax.rsqrt(deg)
    p = (p_ref[0] + p_ref[1])[:N]
    x = x_ref[...]
    agg = p * dinv[:, None] + x * (dinv * dinv)[:, None]
    h = lax.dot_general(agg, w_ref[...], (((1,), (1,)), ((), ())),
                        preferred_element_type=jnp.float32)
    h = jnp.maximum(h, 0.0)
    mean = jnp.mean(h, axis=0)
    var = jnp.mean((h - mean) ** 2, axis=0)
    hn = (h - mean) * lax.rsqrt(var + 1e-5) * g_ref[0] + b_ref[0]
    o_ref[...] = jnp.tanh(hn) + x


def kernel(x, edge_index, W, gamma, beta):
    row = edge_index[0].astype(jnp.int32)
    col = edge_index[1].astype(jnp.int32)

    hist = _sc_hist(row)

    y_pad = pl.pallas_call(
        _tc_scale_body,
        out_shape=jax.ShapeDtypeStruct((NPAD, D), jnp.float32),
    )(hist, x)

    pad = N + jnp.arange(EPAD - E, dtype=jnp.int32) % (NPAD - N)
    col_p = jnp.concatenate([col, pad]).reshape(NW, NCH, CH)
    row_p = jnp.concatenate([row, pad]).reshape(NW, NCH, CH)
    parts = _sc_spmm(y_pad, col_p, row_p)

    out = pl.pallas_call(
        _tc_final_body,
        out_shape=jax.ShapeDtypeStruct((N, D), jnp.float32),
    )(parts, x, hist, W, gamma.reshape(1, D), beta.reshape(1, D))
    return out


# speedup vs baseline: 1.2438x; 1.2438x over previous
import functools

import jax
import jax.numpy as jnp
from jax import lax
from jax.experimental import pallas as pl
from jax.experimental.pallas import tpu as pltpu
from jax.experimental.pallas import tpu_sc as plsc

N = 10000
E = 320000
D = 128
NPAD = 10240
NW = 32
EW = E // NW
CH = 128
NCH = 80
EPAD = NW * NCH * CH
RPW = NPAD // 16

_mesh = plsc.VectorSubcoreMesh(core_axis_name="c", subcore_axis_name="s")
_sc_params = pltpu.CompilerParams(needs_layout_passes=False)


@functools.partial(
    pl.kernel,
    out_type=jax.ShapeDtypeStruct((NW, NPAD), jnp.int32),
    mesh=_mesh,
    scratch_types=[
        pltpu.VMEM((EW,), jnp.int32),
        pltpu.VMEM((NPAD,), jnp.int32),
    ],
    compiler_params=_sc_params,
)
def _sc_hist(row_hbm, out_hbm, idx_v, hist_v):
    cid = lax.axis_index("c")
    sid = lax.axis_index("s")
    w = cid * 16 + sid
    pltpu.sync_copy(row_hbm.at[pl.ds(w * EW, EW)], idx_v)

    def zero_body(i, c):
        hist_v[pl.ds(i * 16, 16)] = jnp.zeros((16,), jnp.int32)
        return c

    lax.fori_loop(0, NPAD // 16, zero_body, 0)
    ones = jnp.ones((16,), jnp.int32)

    def body(i, c):
        idx = idx_v[pl.ds(i * 16, 16)]
        plsc.addupdate_scatter(hist_v, [idx], ones)
        return c

    lax.fori_loop(0, EW // 16, body, 0)
    pltpu.sync_copy(hist_v, out_hbm.at[w])


@functools.partial(
    pl.kernel,
    out_type=jax.ShapeDtypeStruct((2, NPAD, D), jnp.float32),
    mesh=_mesh,
    scratch_types=[
        pltpu.VMEM((NCH // 2, CH), jnp.int32),
        pltpu.VMEM((NCH // 2, CH), jnp.int32),
        pltpu.VMEM((CH, D), jnp.float32),
        pltpu.VMEM((CH, D), jnp.float32),
        pltpu.VMEM_SHARED((NPAD, D), jnp.float32),
        pltpu.SemaphoreType.DMA,
        pltpu.SemaphoreType.DMA,
        pltpu.SemaphoreType.DMA,
        pltpu.SemaphoreType.DMA,
    ],
    compiler_params=_sc_params,
)
def _sc_spmm(y_hbm, col_hbm, row_hbm, out_hbm, col_v, row_v, rows0, rows1,
             agg_sh, sg0, sg1, ss0, ss1):
    cid = lax.axis_index("c")
    sid = lax.axis_index("s")
    w = cid * 16 + sid
    bufs = (rows0, rows1)
    sgs = (sg0, sg1)
    sss = (ss0, ss1)
    HC = NCH // 2

    def zb(i, c):
        rows0[i // 8, pl.ds((i % 8) * 16, 16)] = jnp.zeros((16,), jnp.float32)
        return c

    lax.fori_loop(0, CH * (D // 16), zb, 0)
    zcps = [
        pltpu.async_copy(rows0, agg_sh.at[pl.ds(sid * RPW + i * CH, CH)], sg0)
        for i in range(RPW // CH)
    ]
    for cp in zcps:
        cp.wait()
    plsc.subcore_barrier()

    for h in range(2):
        pltpu.sync_copy(col_hbm.at[w, pl.ds(h * HC, HC)], col_v)
        pltpu.sync_copy(row_hbm.at[w, pl.ds(h * HC, HC)], row_v)
        g0 = pltpu.async_copy(y_hbm.at[col_v.at[0]], rows0, sg0)
        g1 = pltpu.async_copy(y_hbm.at[col_v.at[1]], rows1, sg1)
        gathers = [g0, g1]

        def pair(g, c):
            for b in range(2):
                j = 2 * g + b
                gathers[b].wait()
                sc = pltpu.async_copy(
                    bufs[b], agg_sh.at[row_v.at[j]], sss[b], add=True)
                sc.wait()

                @pl.when(j + 2 < HC)
                def _():
                    pltpu.async_copy(y_hbm.at[col_v.at[j + 2]], bufs[b],
                                     sgs[b])
            return c

        lax.fori_loop(0, HC // 2, pair, 0)

    plsc.subcore_barrier()
    wcps = [
        pltpu.async_copy(agg_sh.at[pl.ds(sid * RPW + i * CH, CH)],
                         out_hbm.at[cid, pl.ds(sid * RPW + i * CH, CH)], sg0)
        for i in range(RPW // CH)
    ]
    for cp in wcps:
        cp.wait()


def _tc_scale_body(hist_ref, x_ref, y_ref):
    deg = jnp.sum(hist_ref[...], axis=0).astype(jnp.float32) + 1.0
    dinv = lax.rsqrt(deg)
    y_ref[pl.ds(0, N)] = x_ref[...] * dinv[:N, None]
    y_ref[pl.ds(N, NPAD - N)] = jnp.zeros((NPAD - N, D), jnp.float32)


def _tc_final_body(p_ref, x_ref, hist_ref, w_ref, g_ref, b_ref, o_ref):
    deg = (jnp.sum(hist_ref[...], axis=0).astype(jnp.float32) + 1.0)[:N]
    dinv = lax.rsqrt(deg)
    p = (p_ref[0] + p_ref[1])[:N]
    x = x_ref[...]
    agg = p * dinv[:, None] + x * (dinv * dinv)[:, None]
    h = lax.dot_general(agg, w_ref[...], (((1,), (1,)), ((), ())),
                        preferred_element_type=jnp.float32)
    h = jnp.maximum(h, 0.0)
    mean = jnp.mean(h, axis=0)
    var = jnp.mean((h - mean) ** 2, axis=0)
    hn = (h - mean) * lax.rsqrt(var + 1e-5) * g_ref[0] + b_ref[0]
    o_ref[...] = jnp.tanh(hn) + x


def kernel(x, edge_index, W, gamma, beta):
    row = edge_index[0].astype(jnp.int32)
    col = edge_index[1].astype(jnp.int32)

    hist = _sc_hist(row)

    y_pad = pl.pallas_call(
        _tc_scale_body,
        out_shape=jax.ShapeDtypeStruct((NPAD, D), jnp.float32),
    )(hist, x)

    pad = N + jnp.arange(EPAD - E, dtype=jnp.int32) % (NPAD - N)
    col_p = jnp.concatenate([col, pad]).reshape(NW, NCH, CH)
    row_p = jnp.concatenate([row, pad]).reshape(NW, NCH, CH)
    parts = _sc_spmm(y_pad, col_p, row_p)

    out = pl.pallas_call(
        _tc_final_body,
        out_shape=jax.ShapeDtypeStruct((N, D), jnp.float32),
    )(parts, x, hist, W, gamma.reshape(1, D), beta.reshape(1, D))
    return out


# speedup vs baseline: 1.2512x; 1.0060x over previous
import functools

import jax
import jax.numpy as jnp
from jax import lax
from jax.experimental import pallas as pl
from jax.experimental.pallas import tpu as pltpu
from jax.experimental.pallas import tpu_sc as plsc

N = 10000
E = 320000
D = 128
NPAD = 10240
NW = 32
EW = E // NW
CH = 64
NCH = 160
EPAD = NW * NCH * CH
RPW = NPAD // 16

_mesh = plsc.VectorSubcoreMesh(core_axis_name="c", subcore_axis_name="s")
_sc_params = pltpu.CompilerParams(needs_layout_passes=False)


@functools.partial(
    pl.kernel,
    out_type=jax.ShapeDtypeStruct((NW, NPAD), jnp.int32),
    mesh=_mesh,
    scratch_types=[
        pltpu.VMEM((EW,), jnp.int32),
        pltpu.VMEM((NPAD,), jnp.int32),
    ],
    compiler_params=_sc_params,
)
def _sc_hist(row_hbm, out_hbm, idx_v, hist_v):
    cid = lax.axis_index("c")
    sid = lax.axis_index("s")
    w = cid * 16 + sid
    pltpu.sync_copy(row_hbm.at[pl.ds(w * EW, EW)], idx_v)

    def zero_body(i, c):
        hist_v[pl.ds(i * 16, 16)] = jnp.zeros((16,), jnp.int32)
        return c

    lax.fori_loop(0, NPAD // 16, zero_body, 0)
    ones = jnp.ones((16,), jnp.int32)

    def body(i, c):
        idx = idx_v[pl.ds(i * 16, 16)]
        plsc.addupdate_scatter(hist_v, [idx], ones)
        return c

    lax.fori_loop(0, EW // 16, body, 0)
    pltpu.sync_copy(hist_v, out_hbm.at[w])


@functools.partial(
    pl.kernel,
    out_type=jax.ShapeDtypeStruct((2, NPAD, D), jnp.float32),
    mesh=_mesh,
    scratch_types=[
        pltpu.VMEM((NCH // 2, CH), jnp.int32),
        pltpu.VMEM((NCH // 2, CH), jnp.int32),
        pltpu.VMEM((CH, D), jnp.float32),
        pltpu.VMEM((CH, D), jnp.float32),
        pltpu.VMEM((CH, D), jnp.float32),
        pltpu.SemaphoreType.DMA,
        pltpu.SemaphoreType.DMA,
        pltpu.SemaphoreType.DMA,
        pltpu.SemaphoreType.DMA,
        pltpu.SemaphoreType.DMA,
        pltpu.SemaphoreType.DMA,
        pltpu.VMEM_SHARED((NPAD, D), jnp.float32),
    ],
    compiler_params=_sc_params,
)
def _sc_spmm(y_hbm, col_hbm, row_hbm, out_hbm, col_v, row_v, b0, b1, b2,
             sg0, sg1, sg2, ss0, ss1, ss2, agg_sh):
    cid = lax.axis_index("c")
    sid = lax.axis_index("s")
    w = cid * 16 + sid
    bufs = (b0, b1, b2)
    sgs = (sg0, sg1, sg2)
    sss = (ss0, ss1, ss2)
    HC = NCH // 2

    def zb(i, c):
        b0[i // 8, pl.ds((i % 8) * 16, 16)] = jnp.zeros((16,), jnp.float32)
        return c

    lax.fori_loop(0, CH * (D // 16), zb, 0)
    zcps = [
        pltpu.async_copy(b0, agg_sh.at[pl.ds(sid * RPW + i * CH, CH)], sg0)
        for i in range(RPW // CH)
    ]
    for cp in zcps:
        cp.wait()
    plsc.subcore_barrier()

    def gather(j, b):
        pltpu.async_copy(y_hbm.at[col_v.at[j]], bufs[b], sgs[b])

    def wait_gather(b):
        pltpu.make_async_copy(y_hbm.at[col_v.at[0]], bufs[b], sgs[b]).wait()

    def scatter(j, b):
        pltpu.async_copy(bufs[b], agg_sh.at[row_v.at[j]], sss[b], add=True)

    def wait_scatter(b):
        pltpu.make_async_copy(bufs[b], agg_sh.at[row_v.at[0]], sss[b]).wait()

    for h in range(2):
        pltpu.sync_copy(col_hbm.at[w, pl.ds(h * HC, HC)], col_v)
        pltpu.sync_copy(row_hbm.at[w, pl.ds(h * HC, HC)], row_v)
        gather(0, 0)
        gather(1, 1)
        wait_gather(0)
        scatter(0, 0)
        gather(2, 2)

        def trip(q, c):
            for u in range(3):
                j = 3 * q + 1 + u
                b = (1 + u) % 3
                wait_gather(b)
                scatter(j, b)
                nb = (b + 2) % 3
                wait_scatter(nb)

                @pl.when(j + 2 < HC)
                def _():
                    gather(j + 2, nb)
            return c

        lax.fori_loop(0, (HC - 2) // 3, trip, 0)
        b = (HC - 1) % 3
        wait_gather(b)
        scatter(HC - 1, b)
        wait_scatter((HC - 2) % 3)
        wait_scatter((HC - 1) % 3)

    plsc.subcore_barrier()
    wcps = [
        pltpu.async_copy(agg_sh.at[pl.ds(sid * RPW + i * 128, 128)],
                         out_hbm.at[cid, pl.ds(sid * RPW + i * 128, 128)],
                         sg0)
        for i in range(RPW // 128)
    ]
    for cp in wcps:
        cp.wait()


def _tc_scale_body(hist_ref, x_ref, y_ref):
    deg = jnp.sum(hist_ref[...], axis=0).astype(jnp.float32) + 1.0
    dinv = lax.rsqrt(deg)
    y_ref[pl.ds(0, N)] = x_ref[...] * dinv[:N, None]
    y_ref[pl.ds(N, NPAD - N)] = jnp.zeros((NPAD - N, D), jnp.float32)


def _tc_final_body(p_ref, x_ref, hist_ref, w_ref, g_ref, b_ref, o_ref):
    deg = (jnp.sum(hist_ref[...], axis=0).astype(jnp.float32) + 1.0)[:N]
    dinv = lax.rsqrt(deg)
    p = (p_ref[0] + p_ref[1])[:N]
    x = x_ref[...]
    agg = p * dinv[:, None] + x * (dinv * dinv)[:, None]
    h = lax.dot_general(agg, w_ref[...], (((1,), (1,)), ((), ())),
                        preferred_element_type=jnp.float32)
    h = jnp.maximum(h, 0.0)
    mean = jnp.mean(h, axis=0)
    var = jnp.mean((h - mean) ** 2, axis=0)
    hn = (h - mean) * lax.rsqrt(var + 1e-5) * g_ref[0] + b_ref[0]
    o_ref[...] = jnp.tanh(hn) + x


def kernel(x, edge_index, W, gamma, beta):
    row = edge_index[0].astype(jnp.int32)
    col = edge_index[1].astype(jnp.int32)

    hist = _sc_hist(row)

    y_pad = pl.pallas_call(
        _tc_scale_body,
        out_shape=jax.ShapeDtypeStruct((NPAD, D), jnp.float32),
    )(hist, x)

    pad = N + jnp.arange(EPAD - E, dtype=jnp.int32) % (NPAD - N)
    col_p = jnp.concatenate([col, pad]).reshape(NW, NCH, CH)
    row_p = jnp.concatenate([row, pad]).reshape(NW, NCH, CH)
    parts = _sc_spmm(y_pad, col_p, row_p)

    out = pl.pallas_call(
        _tc_final_body,
        out_shape=jax.ShapeDtypeStruct((N, D), jnp.float32),
    )(parts, x, hist, W, gamma.reshape(1, D), beta.reshape(1, D))
    return out
